# baseline (speedup 1.0000x reference)
import functools

import jax
import jax.numpy as jnp
from jax import lax
from jax.experimental import pallas as pl
from jax.experimental.pallas import tpu as pltpu
from jax.experimental.pallas import tpu_sc as plsc

E = 8
TOP_K = 2
D_MODEL = 768
D_FF = 2048
CAPACITY = 640
TOKENS = 2048

NROWS = E * CAPACITY + 8
TRASH = E * CAPACITY
NW = 32
CHUNK_T = TOKENS // NW
FF_T = 2048
N_FT = D_FF // FF_T
D_PK = D_MODEL // 2


def _router_kernel(x_ref, rw_ref, sb0_ref, sb1_ref, sg0_ref, sg1_ref,
                   w0_ref, w1_ref, cnt_ref, xp_ref):
    x = x_ref[...]
    rw = rw_ref[...]
    x16 = x.astype(jnp.bfloat16)
    xlb = jax.lax.bitcast_convert_type(x16[:, :D_PK], jnp.uint16)
    xhb = jax.lax.bitcast_convert_type(x16[:, D_PK:], jnp.uint16)
    xp_ref[...] = jax.lax.bitcast_convert_type(
        (xhb.astype(jnp.uint32) << 16) | xlb.astype(jnp.uint32), jnp.int32)
    logits = jnp.dot(x.astype(jnp.bfloat16), rw.astype(jnp.bfloat16),
                     preferred_element_type=jnp.float32)
    lane = lax.broadcasted_iota(jnp.int32, (TOKENS, E), 1)
    m0 = jnp.max(logits, axis=1, keepdims=True)
    i0 = jnp.min(jnp.where(logits == m0, lane, E), axis=1, keepdims=True)
    masked = jnp.where(lane == i0, -jnp.inf, logits)
    m1 = jnp.max(masked, axis=1, keepdims=True)
    i1 = jnp.min(jnp.where(masked == m1, lane, E), axis=1, keepdims=True)
    a = jnp.exp(m1 - m0)
    w0 = 1.0 / (1.0 + a)
    w1 = a / (1.0 + a)
    oh0 = (lane == i0).astype(jnp.float32)
    oh1 = (lane == i1).astype(jnp.float32)
    s = jnp.concatenate(
        [jnp.zeros((1, E), jnp.float32), (oh0 + oh1)[:-1]], axis=0)
    sh = 1
    while sh < TOKENS:
        s = s + jnp.concatenate(
            [jnp.zeros((sh, E), jnp.float32), s[:-sh]], axis=0)
        sh *= 2
    p0 = jnp.sum(s * oh0, axis=1, keepdims=True)
    p1 = jnp.sum(s * oh1, axis=1, keepdims=True)
    keep0 = p0 < float(CAPACITY)
    keep1 = p1 < float(CAPACITY)
    pc0 = jnp.minimum(p0, float(CAPACITY - 1)).astype(jnp.int32)
    pc1 = jnp.minimum(p1, float(CAPACITY - 1)).astype(jnp.int32)
    sg0 = i0 * CAPACITY + pc0
    sg1 = i1 * CAPACITY + pc1
    sb0 = jnp.where(keep0, sg0, TRASH)
    sb1 = jnp.where(keep1, sg1, TRASH)
    wk0 = jnp.where(keep0, w0, 0.0)
    wk1 = jnp.where(keep1, w1, 0.0)
    shp = (TOKENS, E)
    sb0_ref[...] = jnp.broadcast_to(sb0, shp)
    sb1_ref[...] = jnp.broadcast_to(sb1, shp)
    sg0_ref[...] = jnp.broadcast_to(sg0, shp)
    sg1_ref[...] = jnp.broadcast_to(sg1, shp)
    w0_ref[...] = jnp.broadcast_to(wk0, (TOKENS, 16))
    w1_ref[...] = jnp.broadcast_to(wk1, (TOKENS, 16))
    cnt_ref[...] = (s[TOKENS - 1:TOKENS] + oh0[TOKENS - 1:TOKENS]
                    + oh1[TOKENS - 1:TOKENS]).astype(jnp.int32)


def _run_router(x, router_weight):
    out = jax.ShapeDtypeStruct((TOKENS, E), jnp.int32)
    outf = jax.ShapeDtypeStruct((TOKENS, 16), jnp.float32)
    cnt = jax.ShapeDtypeStruct((1, E), jnp.int32)
    xp = jax.ShapeDtypeStruct((TOKENS, D_PK), jnp.int32)
    return pl.pallas_call(
        _router_kernel,
        out_shape=(out, out, out, out, outf, outf, cnt, xp),
    )(x, router_weight)


_H = CHUNK_T // 2


def _dispatch_body(x_hbm, sb0_hbm, sb1_hbm, buf_hbm,
                   xv0, xv1, i00, i01, i10, i11, sa, sb, s0, s1, s2, s3):
    wid = lax.axis_index("s") * 2 + lax.axis_index("c")
    base = wid * CHUNK_T
    ca = pltpu.async_copy(x_hbm.at[pl.ds(base, _H)], xv0, sa)
    cb = pltpu.async_copy(x_hbm.at[pl.ds(base + _H, _H)], xv1, sb)
    pltpu.sync_copy(sb0_hbm.at[pl.ds(base, _H)], i00)
    pltpu.sync_copy(sb0_hbm.at[pl.ds(base + _H, _H)], i01)
    pltpu.sync_copy(sb1_hbm.at[pl.ds(base, _H)], i10)
    pltpu.sync_copy(sb1_hbm.at[pl.ds(base + _H, _H)], i11)
    ca.wait()
    c0 = pltpu.async_copy(xv0, buf_hbm.at[i00], s0)
    c1 = pltpu.async_copy(xv0, buf_hbm.at[i10], s1)
    cb.wait()
    c2 = pltpu.async_copy(xv1, buf_hbm.at[i01], s2)
    c3 = pltpu.async_copy(xv1, buf_hbm.at[i11], s3)
    c0.wait()
    c1.wait()
    c2.wait()
    c3.wait()


def _run_dispatch(x, sb0, sb1):
    mesh = plsc.VectorSubcoreMesh(core_axis_name="c", subcore_axis_name="s")
    fn = functools.partial(
        pl.kernel,
        out_type=jax.ShapeDtypeStruct((NROWS, D_PK), jnp.int32),
        mesh=mesh,
        scratch_types=[
            pltpu.VMEM((_H, D_PK), jnp.int32),
            pltpu.VMEM((_H, D_PK), jnp.int32),
            pltpu.VMEM((_H,), jnp.int32),
            pltpu.VMEM((_H,), jnp.int32),
            pltpu.VMEM((_H,), jnp.int32),
            pltpu.VMEM((_H,), jnp.int32),
            pltpu.SemaphoreType.DMA,
            pltpu.SemaphoreType.DMA,
            pltpu.SemaphoreType.DMA,
            pltpu.SemaphoreType.DMA,
            pltpu.SemaphoreType.DMA,
            pltpu.SemaphoreType.DMA,
        ],
    )(_dispatch_body)
    return fn(x, sb0, sb1)


def _ffn_kernel(buf_ref, gg_ref, gu_ref, dw_ref, y_ref):
    f = pl.program_id(1)
    xw = buf_ref[...]
    xl = jax.lax.bitcast_convert_type(xw << 16, jnp.float32)
    xh = jax.lax.bitcast_convert_type(xw & jnp.int32(-65536), jnp.float32)
    xt = jnp.concatenate([xl, xh], axis=1).astype(jnp.bfloat16)
    gg = gg_ref[0].astype(jnp.bfloat16)
    gu = gu_ref[0].astype(jnp.bfloat16)
    dw = dw_ref[0].astype(jnp.bfloat16)
    g = jnp.dot(xt, gg, preferred_element_type=jnp.float32)
    u = jnp.dot(xt, gu, preferred_element_type=jnp.float32)
    act = (g * jax.nn.sigmoid(g) * u).astype(jnp.bfloat16)
    yp = jnp.dot(act, dw, preferred_element_type=jnp.float32)
    del f
    yb = yp.astype(jnp.bfloat16)
    ylb = jax.lax.bitcast_convert_type(yb[:, :D_PK],
                                       jnp.uint16).astype(jnp.uint32)
    yhb = jax.lax.bitcast_convert_type(yb[:, D_PK:],
                                       jnp.uint16).astype(jnp.uint32)
    y_ref[...] = jax.lax.bitcast_convert_type((yhb << 16) | ylb, jnp.int32)


def _run_ffn(buf, gate_up, dw):
    return pl.pallas_call(
        _ffn_kernel,
        grid=(E, N_FT),
        in_specs=[
            pl.BlockSpec((CAPACITY, D_PK), lambda e, f: (e, 0)),
            pl.BlockSpec((1, D_MODEL, FF_T), lambda e, f: (e, 0, f)),
            pl.BlockSpec((1, D_MODEL, FF_T),
                         lambda e, f: (e, 0, f + N_FT)),
            pl.BlockSpec((1, FF_T, D_MODEL), lambda e, f: (e, f, 0)),
        ],
        out_specs=pl.BlockSpec((CAPACITY, D_PK), lambda e, f: (e, 0)),
        out_shape=jax.ShapeDtypeStruct((E * CAPACITY, D_PK), jnp.int32),
    )(buf, gate_up, gate_up, dw)


def _combine_body(y_hbm, sg0_hbm, sg1_hbm, w0_hbm, w1_hbm, out_hbm,
                  gv0, gv1, ov, i0v, i1v, w0v, w1v, sa, sb):
    wid = lax.axis_index("s") * 2 + lax.axis_index("c")
    base = wid * CHUNK_T
    pltpu.sync_copy(sg0_hbm.at[pl.ds(base, CHUNK_T)], i0v)
    pltpu.sync_copy(sg1_hbm.at[pl.ds(base, CHUNK_T)], i1v)
    ca = pltpu.async_copy(y_hbm.at[i0v], gv0, sa)
    cb = pltpu.async_copy(y_hbm.at[i1v], gv1, sb)
    pltpu.sync_copy(w0_hbm.at[pl.ds(base, CHUNK_T)], w0v)
    pltpu.sync_copy(w1_hbm.at[pl.ds(base, CHUNK_T)], w1v)
    ca.wait()
    cb.wait()
    mask = jnp.int32(-65536)

    def tok(t, carry):
        a = w0v[t]
        b = w1v[t]
        for j in range(D_PK // 16):
            sl = pl.ds(j * 16, 16)
            p0 = gv0[t, sl]
            p1 = gv1[t, sl]
            l0 = jax.lax.bitcast_convert_type(p0 << 16, jnp.float32)
            l1 = jax.lax.bitcast_convert_type(p1 << 16, jnp.float32)
            h0 = jax.lax.bitcast_convert_type(p0 & mask, jnp.float32)
            h1 = jax.lax.bitcast_convert_type(p1 & mask, jnp.float32)
            ov[t, sl] = a * l0 + b * l1
            ov[t, pl.ds(D_PK + j * 16, 16)] = a * h0 + b * h1
        return carry

    lax.fori_loop(0, CHUNK_T, tok, 0)
    pltpu.sync_copy(ov, out_hbm.at[pl.ds(base, CHUNK_T)])


def _run_combine(y, sg0, sg1, w0, w1):
    mesh = plsc.VectorSubcoreMesh(core_axis_name="c", subcore_axis_name="s")
    fn = functools.partial(
        pl.kernel,
        out_type=jax.ShapeDtypeStruct((TOKENS, D_MODEL), jnp.float32),
        mesh=mesh,
        scratch_types=[
            pltpu.VMEM((CHUNK_T, D_PK), jnp.int32),
            pltpu.VMEM((CHUNK_T, D_PK), jnp.int32),
            pltpu.VMEM((CHUNK_T, D_MODEL), jnp.float32),
            pltpu.VMEM((CHUNK_T,), jnp.int32),
            pltpu.VMEM((CHUNK_T,), jnp.int32),
            pltpu.VMEM((CHUNK_T, 16), jnp.float32),
            pltpu.VMEM((CHUNK_T, 16), jnp.float32),
            pltpu.SemaphoreType.DMA,
            pltpu.SemaphoreType.DMA,
        ],
    )(_combine_body)
    return fn(y, sg0, sg1, w0, w1)


def kernel(x, router_weight, gate_up_weight, down_weight):
    sb0, sb1, sg0, sg1, w0, w1, _cnt, xp = _run_router(x, router_weight)
    buf = _run_dispatch(xp, sb0[:, 0], sb1[:, 0])
    y = _run_ffn(buf, gate_up_weight, down_weight)
    return _run_combine(y, sg0[:, 0], sg1[:, 0], w0, w1)


# speedup vs baseline: 1.0346x; 1.0346x over previous
import functools

import jax
import jax.numpy as jnp
from jax import lax
from jax.experimental import pallas as pl
from jax.experimental.pallas import tpu as pltpu
from jax.experimental.pallas import tpu_sc as plsc

E = 8
TOP_K = 2
D_MODEL = 768
D_FF = 2048
CAPACITY = 640
TOKENS = 2048

NROWS = E * CAPACITY + 8
TRASH = E * CAPACITY
NW = 32
CHUNK_T = TOKENS // NW
FF_T = 2048
N_FT = D_FF // FF_T
D_PK = D_MODEL // 2


def _router_kernel(x_ref, rw_ref, sb0_ref, sb1_ref, sg0_ref, sg1_ref,
                   w0_ref, w1_ref, cnt_ref, xp_ref):
    x = x_ref[...]
    rw = rw_ref[...]
    x16 = x.astype(jnp.bfloat16)
    xlb = jax.lax.bitcast_convert_type(x16[:, :D_PK], jnp.uint16)
    xhb = jax.lax.bitcast_convert_type(x16[:, D_PK:], jnp.uint16)
    xp_ref[...] = jax.lax.bitcast_convert_type(
        (xhb.astype(jnp.uint32) << 16) | xlb.astype(jnp.uint32), jnp.int32)
    logits = jnp.dot(x.astype(jnp.bfloat16), rw.astype(jnp.bfloat16),
                     preferred_element_type=jnp.float32)
    lane = lax.broadcasted_iota(jnp.int32, (TOKENS, E), 1)
    m0 = jnp.max(logits, axis=1, keepdims=True)
    i0 = jnp.min(jnp.where(logits == m0, lane, E), axis=1, keepdims=True)
    masked = jnp.where(lane == i0, -jnp.inf, logits)
    m1 = jnp.max(masked, axis=1, keepdims=True)
    i1 = jnp.min(jnp.where(masked == m1, lane, E), axis=1, keepdims=True)
    a = jnp.exp(m1 - m0)
    w0 = 1.0 / (1.0 + a)
    w1 = a / (1.0 + a)
    oh0 = (lane == i0).astype(jnp.float32)
    oh1 = (lane == i1).astype(jnp.float32)
    s = jnp.concatenate(
        [jnp.zeros((1, E), jnp.float32), (oh0 + oh1)[:-1]], axis=0)
    sh = 1
    while sh < TOKENS:
        s = s + jnp.concatenate(
            [jnp.zeros((sh, E), jnp.float32), s[:-sh]], axis=0)
        sh *= 2
    p0 = jnp.sum(s * oh0, axis=1, keepdims=True)
    p1 = jnp.sum(s * oh1, axis=1, keepdims=True)
    keep0 = p0 < float(CAPACITY)
    keep1 = p1 < float(CAPACITY)
    pc0 = jnp.minimum(p0, float(CAPACITY - 1)).astype(jnp.int32)
    pc1 = jnp.minimum(p1, float(CAPACITY - 1)).astype(jnp.int32)
    sg0 = i0 * CAPACITY + pc0
    sg1 = i1 * CAPACITY + pc1
    sb0 = jnp.where(keep0, sg0, TRASH)
    sb1 = jnp.where(keep1, sg1, TRASH)
    wk0 = jnp.where(keep0, w0, 0.0)
    wk1 = jnp.where(keep1, w1, 0.0)
    shp = (TOKENS, E)
    sb0_ref[...] = jnp.broadcast_to(sb0, shp)
    sb1_ref[...] = jnp.broadcast_to(sb1, shp)
    sg0_ref[...] = jnp.broadcast_to(sg0, shp)
    sg1_ref[...] = jnp.broadcast_to(sg1, shp)
    w0_ref[...] = jnp.broadcast_to(wk0, (TOKENS, 16))
    w1_ref[...] = jnp.broadcast_to(wk1, (TOKENS, 16))
    cnt_ref[...] = (s[TOKENS - 1:TOKENS] + oh0[TOKENS - 1:TOKENS]
                    + oh1[TOKENS - 1:TOKENS]).astype(jnp.int32)


def _run_router(x, router_weight):
    out = jax.ShapeDtypeStruct((TOKENS, E), jnp.int32)
    outf = jax.ShapeDtypeStruct((TOKENS, 16), jnp.float32)
    cnt = jax.ShapeDtypeStruct((1, E), jnp.int32)
    xp = jax.ShapeDtypeStruct((TOKENS, D_PK), jnp.int32)
    return pl.pallas_call(
        _router_kernel,
        out_shape=(out, out, out, out, outf, outf, cnt, xp),
    )(x, router_weight)


_H = CHUNK_T // 2


def _dispatch_body(x_hbm, sb0_hbm, sb1_hbm, buf_hbm,
                   xv0, xv1, i00, i01, i10, i11, sa, sb, s0, s1, s2, s3):
    wid = lax.axis_index("s") * 2 + lax.axis_index("c")
    base = wid * CHUNK_T
    ca = pltpu.async_copy(x_hbm.at[pl.ds(base, _H)], xv0, sa)
    cb = pltpu.async_copy(x_hbm.at[pl.ds(base + _H, _H)], xv1, sb)
    pltpu.sync_copy(sb0_hbm.at[pl.ds(base, _H)], i00)
    pltpu.sync_copy(sb0_hbm.at[pl.ds(base + _H, _H)], i01)
    pltpu.sync_copy(sb1_hbm.at[pl.ds(base, _H)], i10)
    pltpu.sync_copy(sb1_hbm.at[pl.ds(base + _H, _H)], i11)
    ca.wait()
    c0 = pltpu.async_copy(xv0, buf_hbm.at[i00], s0)
    c1 = pltpu.async_copy(xv0, buf_hbm.at[i10], s1)
    cb.wait()
    c2 = pltpu.async_copy(xv1, buf_hbm.at[i01], s2)
    c3 = pltpu.async_copy(xv1, buf_hbm.at[i11], s3)
    c0.wait()
    c1.wait()
    c2.wait()
    c3.wait()


def _run_dispatch(x, sb0, sb1):
    mesh = plsc.VectorSubcoreMesh(core_axis_name="c", subcore_axis_name="s")
    fn = functools.partial(
        pl.kernel,
        out_type=jax.ShapeDtypeStruct((NROWS, D_PK), jnp.int32),
        mesh=mesh,
        scratch_types=[
            pltpu.VMEM((_H, D_PK), jnp.int32),
            pltpu.VMEM((_H, D_PK), jnp.int32),
            pltpu.VMEM((_H,), jnp.int32),
            pltpu.VMEM((_H,), jnp.int32),
            pltpu.VMEM((_H,), jnp.int32),
            pltpu.VMEM((_H,), jnp.int32),
            pltpu.SemaphoreType.DMA,
            pltpu.SemaphoreType.DMA,
            pltpu.SemaphoreType.DMA,
            pltpu.SemaphoreType.DMA,
            pltpu.SemaphoreType.DMA,
            pltpu.SemaphoreType.DMA,
        ],
    )(_dispatch_body)
    return fn(x, sb0, sb1)


def _ffn_kernel(buf_ref, gg_ref, gu_ref, dw_ref, y_ref):
    f = pl.program_id(1)
    xw = buf_ref[...]
    xl = jax.lax.bitcast_convert_type(xw << 16, jnp.float32)
    xh = jax.lax.bitcast_convert_type(xw & jnp.int32(-65536), jnp.float32)
    xt = jnp.concatenate([xl, xh], axis=1).astype(jnp.bfloat16)
    gg = gg_ref[0].astype(jnp.bfloat16)
    gu = gu_ref[0].astype(jnp.bfloat16)
    dw = dw_ref[0].astype(jnp.bfloat16)
    g = jnp.dot(xt, gg, preferred_element_type=jnp.float32)
    u = jnp.dot(xt, gu, preferred_element_type=jnp.float32)
    act = (g * jax.nn.sigmoid(g) * u).astype(jnp.bfloat16)
    yp = jnp.dot(act, dw, preferred_element_type=jnp.float32)

    @pl.when(f == 0)
    def _():
        y_ref[...] = yp

    @pl.when(f != 0)
    def _():
        y_ref[...] += yp


def _run_ffn(buf, gate_up, dw):
    return pl.pallas_call(
        _ffn_kernel,
        grid=(E, N_FT),
        in_specs=[
            pl.BlockSpec((CAPACITY, D_PK), lambda e, f: (e, 0)),
            pl.BlockSpec((1, D_MODEL, FF_T), lambda e, f: (e, 0, f)),
            pl.BlockSpec((1, D_MODEL, FF_T),
                         lambda e, f: (e, 0, f + N_FT)),
            pl.BlockSpec((1, FF_T, D_MODEL), lambda e, f: (e, f, 0)),
        ],
        out_specs=pl.BlockSpec((CAPACITY, D_MODEL), lambda e, f: (e, 0)),
        out_shape=jax.ShapeDtypeStruct((E * CAPACITY, D_MODEL), jnp.float32),
    )(buf, gate_up, gate_up, dw)


def _combine_body(y_hbm, sg0_hbm, sg1_hbm, w0_hbm, w1_hbm, out_hbm,
                  gv0, gv1, i0v, i1v, w0v, w1v, sa, sb):
    wid = lax.axis_index("s") * 2 + lax.axis_index("c")
    base = wid * CHUNK_T
    pltpu.sync_copy(sg0_hbm.at[pl.ds(base, CHUNK_T)], i0v)
    pltpu.sync_copy(sg1_hbm.at[pl.ds(base, CHUNK_T)], i1v)
    ca = pltpu.async_copy(y_hbm.at[i0v], gv0, sa)
    cb = pltpu.async_copy(y_hbm.at[i1v], gv1, sb)
    pltpu.sync_copy(w0_hbm.at[pl.ds(base, CHUNK_T)], w0v)
    pltpu.sync_copy(w1_hbm.at[pl.ds(base, CHUNK_T)], w1v)
    ca.wait()
    cb.wait()

    def tok(t, carry):
        a = w0v[t]
        b = w1v[t]
        for j in range(D_MODEL // 16):
            sl = pl.ds(j * 16, 16)
            gv0[t, sl] = a * gv0[t, sl] + b * gv1[t, sl]
        return carry

    lax.fori_loop(0, CHUNK_T, tok, 0)
    pltpu.sync_copy(gv0, out_hbm.at[pl.ds(base, CHUNK_T)])


def _run_combine(y, sg0, sg1, w0, w1):
    mesh = plsc.VectorSubcoreMesh(core_axis_name="c", subcore_axis_name="s")
    fn = functools.partial(
        pl.kernel,
        out_type=jax.ShapeDtypeStruct((TOKENS, D_MODEL), jnp.float32),
        mesh=mesh,
        scratch_types=[
            pltpu.VMEM((CHUNK_T, D_MODEL), jnp.float32),
            pltpu.VMEM((CHUNK_T, D_MODEL), jnp.float32),
            pltpu.VMEM((CHUNK_T,), jnp.int32),
            pltpu.VMEM((CHUNK_T,), jnp.int32),
            pltpu.VMEM((CHUNK_T, 16), jnp.float32),
            pltpu.VMEM((CHUNK_T, 16), jnp.float32),
            pltpu.SemaphoreType.DMA,
            pltpu.SemaphoreType.DMA,
        ],
    )(_combine_body)
    return fn(y, sg0, sg1, w0, w1)


def kernel(x, router_weight, gate_up_weight, down_weight):
    sb0, sb1, sg0, sg1, w0, w1, _cnt, xp = _run_router(x, router_weight)
    buf = _run_dispatch(xp, sb0[:, 0], sb1[:, 0])
    y = _run_ffn(buf, gate_up_weight, down_weight)
    return _run_combine(y, sg0[:, 0], sg1[:, 0], w0, w1)


# speedup vs baseline: 1.0363x; 1.0017x over previous
import functools

import jax
import jax.numpy as jnp
from jax import lax
from jax.experimental import pallas as pl
from jax.experimental.pallas import tpu as pltpu
from jax.experimental.pallas import tpu_sc as plsc

E = 8
TOP_K = 2
D_MODEL = 768
D_FF = 2048
CAPACITY = 640
TOKENS = 2048

NROWS = E * CAPACITY + 8
TRASH = E * CAPACITY
NW = 32
CHUNK_T = TOKENS // NW
FF_T = 2048
N_FT = D_FF // FF_T
D_PK = D_MODEL // 2


def _router_kernel(x_ref, rw_ref, sb0_ref, sb1_ref, sg0_ref, sg1_ref,
                   w0_ref, w1_ref, xp_ref):
    x = x_ref[...]
    rw = rw_ref[...]
    x16 = x.astype(jnp.bfloat16)
    xlb = jax.lax.bitcast_convert_type(x16[:, :D_PK], jnp.uint16)
    xhb = jax.lax.bitcast_convert_type(x16[:, D_PK:], jnp.uint16)
    xp_ref[...] = jax.lax.bitcast_convert_type(
        (xhb.astype(jnp.uint32) << 16) | xlb.astype(jnp.uint32), jnp.int32)
    logits = jnp.dot(x.astype(jnp.bfloat16), rw.astype(jnp.bfloat16),
                     preferred_element_type=jnp.float32)
    lane = lax.broadcasted_iota(jnp.int32, (TOKENS, E), 1)
    m0 = jnp.max(logits, axis=1, keepdims=True)
    i0 = jnp.min(jnp.where(logits == m0, lane, E), axis=1, keepdims=True)
    masked = jnp.where(lane == i0, -jnp.inf, logits)
    m1 = jnp.max(masked, axis=1, keepdims=True)
    i1 = jnp.min(jnp.where(masked == m1, lane, E), axis=1, keepdims=True)
    a = jnp.exp(m1 - m0)
    w0 = 1.0 / (1.0 + a)
    w1 = a / (1.0 + a)
    oh0 = (lane == i0).astype(jnp.float32)
    oh1 = (lane == i1).astype(jnp.float32)
    s = jnp.concatenate(
        [jnp.zeros((1, E), jnp.float32), (oh0 + oh1)[:-1]], axis=0)
    sh = 1
    while sh < TOKENS:
        s = s + jnp.concatenate(
            [jnp.zeros((sh, E), jnp.float32), s[:-sh]], axis=0)
        sh *= 2
    p0 = jnp.sum(s * oh0, axis=1, keepdims=True)
    p1 = jnp.sum(s * oh1, axis=1, keepdims=True)
    keep0 = p0 < float(CAPACITY)
    keep1 = p1 < float(CAPACITY)
    pc0 = jnp.minimum(p0, float(CAPACITY - 1)).astype(jnp.int32)
    pc1 = jnp.minimum(p1, float(CAPACITY - 1)).astype(jnp.int32)
    sg0 = i0 * CAPACITY + pc0
    sg1 = i1 * CAPACITY + pc1
    sb0 = jnp.where(keep0, sg0, TRASH)
    sb1 = jnp.where(keep1, sg1, TRASH)
    wk0 = jnp.where(keep0, w0, 0.0)
    wk1 = jnp.where(keep1, w1, 0.0)
    shp = (TOKENS, E)
    sb0_ref[...] = jnp.broadcast_to(sb0, shp)
    sb1_ref[...] = jnp.broadcast_to(sb1, shp)
    sg0_ref[...] = jnp.broadcast_to(sg0, shp)
    sg1_ref[...] = jnp.broadcast_to(sg1, shp)
    w0_ref[...] = jnp.broadcast_to(wk0, (TOKENS, 16))
    w1_ref[...] = jnp.broadcast_to(wk1, (TOKENS, 16))


def _run_router(x, router_weight):
    out = jax.ShapeDtypeStruct((TOKENS, E), jnp.int32)
    outf = jax.ShapeDtypeStruct((TOKENS, 16), jnp.float32)
    xp = jax.ShapeDtypeStruct((TOKENS, D_PK), jnp.int32)
    return pl.pallas_call(
        _router_kernel,
        out_shape=(out, out, out, out, outf, outf, xp),
    )(x, router_weight)


_H = CHUNK_T // 2


def _dispatch_body(x_hbm, sb0_hbm, sb1_hbm, buf_hbm,
                   xv0, xv1, i00, i01, i10, i11, sa, sb, s0, s1, s2, s3):
    wid = lax.axis_index("s") * 2 + lax.axis_index("c")
    base = wid * CHUNK_T
    ca = pltpu.async_copy(x_hbm.at[pl.ds(base, _H)], xv0, sa)
    cb = pltpu.async_copy(x_hbm.at[pl.ds(base + _H, _H)], xv1, sb)
    pltpu.sync_copy(sb0_hbm.at[pl.ds(base, _H)], i00)
    pltpu.sync_copy(sb0_hbm.at[pl.ds(base + _H, _H)], i01)
    pltpu.sync_copy(sb1_hbm.at[pl.ds(base, _H)], i10)
    pltpu.sync_copy(sb1_hbm.at[pl.ds(base + _H, _H)], i11)
    ca.wait()
    c0 = pltpu.async_copy(xv0, buf_hbm.at[i00], s0)
    c1 = pltpu.async_copy(xv0, buf_hbm.at[i10], s1)
    cb.wait()
    c2 = pltpu.async_copy(xv1, buf_hbm.at[i01], s2)
    c3 = pltpu.async_copy(xv1, buf_hbm.at[i11], s3)
    c0.wait()
    c1.wait()
    c2.wait()
    c3.wait()


def _run_dispatch(x, sb0, sb1):
    mesh = plsc.VectorSubcoreMesh(core_axis_name="c", subcore_axis_name="s")
    fn = functools.partial(
        pl.kernel,
        out_type=jax.ShapeDtypeStruct((NROWS, D_PK), jnp.int32),
        mesh=mesh,
        scratch_types=[
            pltpu.VMEM((_H, D_PK), jnp.int32),
            pltpu.VMEM((_H, D_PK), jnp.int32),
            pltpu.VMEM((_H,), jnp.int32),
            pltpu.VMEM((_H,), jnp.int32),
            pltpu.VMEM((_H,), jnp.int32),
            pltpu.VMEM((_H,), jnp.int32),
            pltpu.SemaphoreType.DMA,
            pltpu.SemaphoreType.DMA,
            pltpu.SemaphoreType.DMA,
            pltpu.SemaphoreType.DMA,
            pltpu.SemaphoreType.DMA,
            pltpu.SemaphoreType.DMA,
        ],
    )(_dispatch_body)
    return fn(x, sb0, sb1)


def _ffn_kernel(buf_ref, gg_ref, gu_ref, dw_ref, y_ref):
    f = pl.program_id(1)
    xw = buf_ref[...]
    xl = jax.lax.bitcast_convert_type(xw << 16, jnp.float32)
    xh = jax.lax.bitcast_convert_type(xw & jnp.int32(-65536), jnp.float32)
    xt = jnp.concatenate([xl, xh], axis=1).astype(jnp.bfloat16)
    gg = gg_ref[0].astype(jnp.bfloat16)
    gu = gu_ref[0].astype(jnp.bfloat16)
    dw = dw_ref[0].astype(jnp.bfloat16)
    g = jnp.dot(xt, gg, preferred_element_type=jnp.float32)
    u = jnp.dot(xt, gu, preferred_element_type=jnp.float32)
    act = (g * jax.nn.sigmoid(g) * u).astype(jnp.bfloat16)
    yp = jnp.dot(act, dw, preferred_element_type=jnp.float32)

    @pl.when(f == 0)
    def _():
        y_ref[...] = yp

    @pl.when(f != 0)
    def _():
        y_ref[...] += yp


def _run_ffn(buf, gate_up, dw):
    return pl.pallas_call(
        _ffn_kernel,
        grid=(E, N_FT),
        in_specs=[
            pl.BlockSpec((CAPACITY, D_PK), lambda e, f: (e, 0)),
            pl.BlockSpec((1, D_MODEL, FF_T), lambda e, f: (e, 0, f)),
            pl.BlockSpec((1, D_MODEL, FF_T),
                         lambda e, f: (e, 0, f + N_FT)),
            pl.BlockSpec((1, FF_T, D_MODEL), lambda e, f: (e, f, 0)),
        ],
        out_specs=pl.BlockSpec((CAPACITY, D_MODEL), lambda e, f: (e, 0)),
        out_shape=jax.ShapeDtypeStruct((E * CAPACITY, D_MODEL), jnp.float32),
    )(buf, gate_up, gate_up, dw)


def _combine_body(y_hbm, sg0_hbm, sg1_hbm, w0_hbm, w1_hbm, out_hbm,
                  gv0, gv1, i0v, i1v, w0v, w1v, sa, sb):
    wid = lax.axis_index("s") * 2 + lax.axis_index("c")
    base = wid * CHUNK_T
    pltpu.sync_copy(sg0_hbm.at[pl.ds(base, CHUNK_T)], i0v)
    pltpu.sync_copy(sg1_hbm.at[pl.ds(base, CHUNK_T)], i1v)
    ca = pltpu.async_copy(y_hbm.at[i0v], gv0, sa)
    cb = pltpu.async_copy(y_hbm.at[i1v], gv1, sb)
    pltpu.sync_copy(w0_hbm.at[pl.ds(base, CHUNK_T)], w0v)
    pltpu.sync_copy(w1_hbm.at[pl.ds(base, CHUNK_T)], w1v)
    ca.wait()
    cb.wait()

    def tok(t, carry):
        a = w0v[t]
        b = w1v[t]
        for j in range(D_MODEL // 16):
            sl = pl.ds(j * 16, 16)
            gv0[t, sl] = a * gv0[t, sl] + b * gv1[t, sl]
        return carry

    lax.fori_loop(0, CHUNK_T, tok, 0)
    pltpu.sync_copy(gv0, out_hbm.at[pl.ds(base, CHUNK_T)])


def _run_combine(y, sg0, sg1, w0, w1):
    mesh = plsc.VectorSubcoreMesh(core_axis_name="c", subcore_axis_name="s")
    fn = functools.partial(
        pl.kernel,
        out_type=jax.ShapeDtypeStruct((TOKENS, D_MODEL), jnp.float32),
        mesh=mesh,
        scratch_types=[
            pltpu.VMEM((CHUNK_T, D_MODEL), jnp.float32),
            pltpu.VMEM((CHUNK_T, D_MODEL), jnp.float32),
            pltpu.VMEM((CHUNK_T,), jnp.int32),
            pltpu.VMEM((CHUNK_T,), jnp.int32),
            pltpu.VMEM((CHUNK_T, 16), jnp.float32),
            pltpu.VMEM((CHUNK_T, 16), jnp.float32),
            pltpu.SemaphoreType.DMA,
            pltpu.SemaphoreType.DMA,
        ],
    )(_combine_body)
    return fn(y, sg0, sg1, w0, w1)


def kernel(x, router_weight, gate_up_weight, down_weight):
    sb0, sb1, sg0, sg1, w0, w1, xp = _run_router(x, router_weight)
    buf = _run_dispatch(xp, sb0[:, 0], sb1[:, 0])
    y = _run_ffn(buf, gate_up_weight, down_weight)
    return _run_combine(y, sg0[:, 0], sg1[:, 0], w0, w1)
